# Initial kernel scaffold; baseline (speedup 1.0000x reference)
import jax
import jax.numpy as jnp
from jax.experimental import pallas as pl


def kernel(x, edge_index, edge_attr, batch, Wg, bg, W1, b1, W2, b2, W3, b3):
    raise NotImplementedError("write your pallas kernel here")


# speedup vs baseline: 31.5365x; 31.5365x over previous
import dataclasses
import functools

import jax
import jax.numpy as jnp
from jax import lax
from jax.experimental import pallas as pl
from jax.experimental.pallas import tpu as pltpu
from jax.experimental.pallas import tpu_sc as plsc

N = 32768
E = 524288
BS = 1024
NPG = 32
IN_CH = 128
HID = 64
HHID = HID // 2
NC = 10

NSUB = 16
NCORE = 2
CH = 128
ROWS_A = (E // CH) // (NCORE * NSUB)
ROWS_C = (E // CH) // NSUB
ROWS_B = 32
NPT = N // NSUB

_sc_params = pltpu.CompilerParams()
if "needs_layout_passes" in pltpu.CompilerParams.__dataclass_fields__:
    _sc_params = dataclasses.replace(_sc_params, needs_layout_passes=False)
if "use_tc_tiling_on_sc" in pltpu.CompilerParams.__dataclass_fields__:
    _sc_params = dataclasses.replace(_sc_params, use_tc_tiling_on_sc=False)

_mesh = plsc.VectorSubcoreMesh(core_axis_name="c", subcore_axis_name="s")


@functools.partial(
    pl.kernel,
    out_type=jax.ShapeDtypeStruct((NCORE, N), jnp.float32),
    mesh=_mesh,
    scratch_types=[
        pltpu.VMEM((ROWS_A, CH), jnp.int32),
        pltpu.VMEM((ROWS_A, CH), jnp.float32),
        pltpu.VMEM((NPT,), jnp.float32),
        pltpu.VMEM_SHARED((N,), jnp.float32),
    ],
)
def _deg_partials(col_hbm, w_hbm, out_hbm, colv, wv, zbuf, deg_sh):
    c = lax.axis_index("c")
    s = lax.axis_index("s")
    tid = c * jnp.int32(NSUB) + s

    @pl.loop(0, NPT // 16)
    def _zero(i):
        zbuf[pl.ds(i * jnp.int32(16), 16)] = jnp.zeros((16,), jnp.float32)

    pltpu.sync_copy(zbuf, deg_sh.at[pl.ds(s * jnp.int32(NPT), NPT)])
    plsc.subcore_barrier()

    pltpu.sync_copy(col_hbm.at[pl.ds(tid * jnp.int32(ROWS_A), ROWS_A)], colv)
    pltpu.sync_copy(w_hbm.at[pl.ds(tid * jnp.int32(ROWS_A), ROWS_A)], wv)

    @pl.loop(0, ROWS_A)
    def _acc(j):
        pltpu.sync_copy(wv.at[j], deg_sh.at[colv.at[j]], add=True)

    plsc.subcore_barrier()
    pltpu.sync_copy(deg_sh.at[pl.ds(s * jnp.int32(NPT), NPT)],
                    out_hbm.at[c, pl.ds(s * jnp.int32(NPT), NPT)])


@functools.partial(
    pl.kernel,
    out_type=[jax.ShapeDtypeStruct((N, HHID), jnp.float32),
              jax.ShapeDtypeStruct((N, HHID), jnp.float32)],
    mesh=_mesh,
    scratch_types=[
        pltpu.VMEM((ROWS_B, CH), jnp.int32),
        pltpu.VMEM((ROWS_B, CH), jnp.int32),
        pltpu.VMEM((ROWS_B * CH,), jnp.float32),
        pltpu.VMEM((CH, HHID), jnp.float32),
        pltpu.VMEM((CH, HHID), jnp.float32),
        pltpu.VMEM((64, HHID), jnp.float32),
        pltpu.VMEM_SHARED((N, HHID), jnp.float32),
        pltpu.SemaphoreType.DMA,
        pltpu.SemaphoreType.DMA,
    ],
    compiler_params=_sc_params,
)
def _edge_scatter(row_hbm, col_hbm, w_hbm, g0_hbm, g1_hbm, p0_hbm, p1_hbm,
                  rowv, colv, wblk, G0, G1, zbuf, p_sh, sem0, sem1):
    c = lax.axis_index("c")
    s = lax.axis_index("s")

    @pl.loop(0, (64 * HHID) // 16)
    def _zero(i):
        r = i // jnp.int32(HHID // 16)
        k = (i % jnp.int32(HHID // 16)) * jnp.int32(16)
        zbuf[r, pl.ds(k, 16)] = jnp.zeros((16,), jnp.float32)

    @pl.loop(0, NPT // 64)
    def _zero_sh(i):
        pltpu.sync_copy(zbuf, p_sh.at[pl.ds(s * jnp.int32(NPT) + i * jnp.int32(64), 64)])

    plsc.subcore_barrier()

    def scale_scatter(G, j):
        @pl.loop(0, CH)
        def _sc(r):
            wb = plsc.load_gather(
                wblk, [jnp.full((16,), j * jnp.int32(CH) + r, jnp.int32)])
            a = G[r, pl.ds(0, 16)] * wb
            b = G[r, pl.ds(16, 16)] * wb
            G[r, pl.ds(0, 16)] = a
            G[r, pl.ds(16, 16)] = b
        pltpu.sync_copy(G, p_sh.at[colv.at[j]], add=True)

    def run(g_hbm):
        @pl.loop(0, ROWS_C // ROWS_B)
        def _blk(b):
            base = (s * jnp.int32(ROWS_C) + b * jnp.int32(ROWS_B))
            pltpu.sync_copy(row_hbm.at[pl.ds(base, ROWS_B)], rowv)
            pltpu.sync_copy(col_hbm.at[pl.ds(base, ROWS_B)], colv)
            pltpu.sync_copy(w_hbm.at[pl.ds(base * jnp.int32(CH), ROWS_B * CH)],
                            wblk)
            pltpu.make_async_copy(g_hbm.at[rowv.at[0]], G0, sem0).start()

            @pl.loop(0, ROWS_B // 2)
            def _main(jj):
                j0 = jj * jnp.int32(2)
                pltpu.make_async_copy(g_hbm.at[rowv.at[j0]], G0, sem0).wait()
                pltpu.make_async_copy(
                    g_hbm.at[rowv.at[j0 + 1]], G1, sem1).start()
                scale_scatter(G0, j0)
                pltpu.make_async_copy(
                    g_hbm.at[rowv.at[j0 + 1]], G1, sem1).wait()

                @pl.when(jj < ROWS_B // 2 - 1)
                def _next():
                    pltpu.make_async_copy(
                        g_hbm.at[rowv.at[j0 + 2]], G0, sem0).start()

                scale_scatter(G1, j0 + 1)

    @pl.when(c == 0)
    def _c0():
        run(g0_hbm)

    @pl.when(c == 1)
    def _c1():
        run(g1_hbm)

    plsc.subcore_barrier()

    @pl.when(c == 0)
    def _w0():
        pltpu.sync_copy(p_sh.at[pl.ds(s * jnp.int32(NPT), NPT)],
                        p0_hbm.at[pl.ds(s * jnp.int32(NPT), NPT)])

    @pl.when(c == 1)
    def _w1():
        pltpu.sync_copy(p_sh.at[pl.ds(s * jnp.int32(NPT), NPT)],
                        p1_hbm.at[pl.ds(s * jnp.int32(NPT), NPT)])


_NB = 2048


def _h_body(x_ref, wg_ref, h_ref):
    h_ref[...] = jnp.dot(x_ref[...], wg_ref[...],
                         preferred_element_type=jnp.float32)


def _prep_body(h_ref, part_ref, g0_ref, g1_ref, dinv_ref):
    deg = 1.0 + part_ref[0, :] + part_ref[1, :]
    dinv = jnp.where(deg > 0, lax.rsqrt(jnp.maximum(deg, 1e-12)), 0.0)
    g = h_ref[...] * dinv[:, None]
    g0_ref[...] = g[:, :HHID]
    g1_ref[...] = g[:, HHID:]
    dinv_ref[...] = dinv


def _agg_body(p0_ref, p1_ref, g0_ref, g1_ref, dinv_ref, bg_ref, agg_ref):
    p = jnp.concatenate([p0_ref[...], p1_ref[...]], axis=1)
    g = jnp.concatenate([g0_ref[...], g1_ref[...]], axis=1)
    agg_ref[...] = dinv_ref[...][:, None] * (p + g) + bg_ref[...][None, :]


def _mlp_body(z_ref, w1_ref, b1_ref, w2_ref, b2_ref, w3_ref, b3_ref, o_ref):
    z1 = jnp.dot(z_ref[...], w1_ref[...], preferred_element_type=jnp.float32)
    z1 = jnp.maximum(z1 + b1_ref[...][None, :], 0.0)
    z2 = jnp.dot(z1, w2_ref[...], preferred_element_type=jnp.float32)
    z2 = jnp.maximum(z2 + b2_ref[...][None, :], 0.0)
    z3 = jnp.dot(z2, w3_ref[...], preferred_element_type=jnp.float32)
    z3 = z3 + b3_ref[...][None, :]
    m = jnp.max(z3, axis=-1, keepdims=True)
    e = jnp.exp(z3 - m)
    o_ref[...] = e / jnp.sum(e, axis=-1, keepdims=True)


def _matmul_h(x, Wg):
    return pl.pallas_call(
        _h_body,
        grid=(N // _NB,),
        in_specs=[pl.BlockSpec((_NB, IN_CH), lambda i: (i, 0)),
                  pl.BlockSpec((IN_CH, HID), lambda i: (0, 0))],
        out_specs=pl.BlockSpec((_NB, HID), lambda i: (i, 0)),
        out_shape=jax.ShapeDtypeStruct((N, HID), jnp.float32),
    )(x, Wg)


def _prep(h, partials):
    return pl.pallas_call(
        _prep_body,
        grid=(N // _NB,),
        in_specs=[pl.BlockSpec((_NB, HID), lambda i: (i, 0)),
                  pl.BlockSpec((NCORE, _NB), lambda i: (0, i))],
        out_specs=[pl.BlockSpec((_NB, HHID), lambda i: (i, 0)),
                   pl.BlockSpec((_NB, HHID), lambda i: (i, 0)),
                   pl.BlockSpec((_NB,), lambda i: (i,))],
        out_shape=[jax.ShapeDtypeStruct((N, HHID), jnp.float32),
                   jax.ShapeDtypeStruct((N, HHID), jnp.float32),
                   jax.ShapeDtypeStruct((N,), jnp.float32)],
    )(h, partials)


def _aggregate(p0, p1, g0, g1, dinv, bg):
    return pl.pallas_call(
        _agg_body,
        grid=(N // _NB,),
        in_specs=[pl.BlockSpec((_NB, HHID), lambda i: (i, 0)),
                  pl.BlockSpec((_NB, HHID), lambda i: (i, 0)),
                  pl.BlockSpec((_NB, HHID), lambda i: (i, 0)),
                  pl.BlockSpec((_NB, HHID), lambda i: (i, 0)),
                  pl.BlockSpec((_NB,), lambda i: (i,)),
                  pl.BlockSpec((HID,), lambda i: (0,))],
        out_specs=pl.BlockSpec((_NB, HID), lambda i: (i, 0)),
        out_shape=jax.ShapeDtypeStruct((N, HID), jnp.float32),
    )(p0, p1, g0, g1, dinv, bg)


def _mlp(z, W1, b1, W2, b2, W3, b3):
    gb = 256
    K = NPG * HID
    return pl.pallas_call(
        _mlp_body,
        grid=(BS // gb,),
        in_specs=[pl.BlockSpec((gb, K), lambda i: (i, 0)),
                  pl.BlockSpec((K, HID), lambda i: (0, 0)),
                  pl.BlockSpec((HID,), lambda i: (0,)),
                  pl.BlockSpec((HID, HHID), lambda i: (0, 0)),
                  pl.BlockSpec((HHID,), lambda i: (0,)),
                  pl.BlockSpec((HHID, NC), lambda i: (0, 0)),
                  pl.BlockSpec((NC,), lambda i: (0,))],
        out_specs=pl.BlockSpec((gb, NC), lambda i: (i, 0)),
        out_shape=jax.ShapeDtypeStruct((BS, NC), jnp.float32),
    )(z, W1, b1, W2, b2, W3, b3)


def kernel(x, edge_index, edge_attr, batch, Wg, bg, W1, b1, W2, b2, W3, b3):
    del batch
    with jax.enable_x64(False):
        x = x.astype(jnp.float32)
        ei = edge_index.astype(jnp.int32)
        row2d = ei[0].reshape(E // CH, CH)
        col2d = ei[1].reshape(E // CH, CH)
        w = edge_attr.astype(jnp.float32)
        w2d = w.reshape(E // CH, CH)

        partials = _deg_partials(col2d, w2d)
        h = _matmul_h(x, Wg)
        g0, g1, dinv = _prep(h, partials)
        p0, p1 = _edge_scatter(row2d, col2d, w, g0, g1)
        agg = _aggregate(p0, p1, g0, g1, dinv, bg)
        z = agg.reshape(BS, NPG * HID)
        return _mlp(z, W1, b1, W2, b2, W3, b3)


# speedup vs baseline: 37.8110x; 1.1990x over previous
import dataclasses
import functools

import jax
import jax.numpy as jnp
from jax import lax
from jax.experimental import pallas as pl
from jax.experimental.pallas import tpu as pltpu
from jax.experimental.pallas import tpu_sc as plsc

N = 32768
E = 524288
BS = 1024
NPG = 32
IN_CH = 128
HID = 64
HHID = HID // 2
NC = 10

NSUB = 16
NCORE = 2
CH = 128
ROWS_A = (E // CH) // (NCORE * NSUB)
ROWS_C = (E // CH) // NSUB
ROWS_B = 64
NPT = N // NSUB

_sc_params = pltpu.CompilerParams()
if "needs_layout_passes" in pltpu.CompilerParams.__dataclass_fields__:
    _sc_params = dataclasses.replace(_sc_params, needs_layout_passes=False)
if "use_tc_tiling_on_sc" in pltpu.CompilerParams.__dataclass_fields__:
    _sc_params = dataclasses.replace(_sc_params, use_tc_tiling_on_sc=False)

_mesh = plsc.VectorSubcoreMesh(core_axis_name="c", subcore_axis_name="s")


@functools.partial(
    pl.kernel,
    out_type=jax.ShapeDtypeStruct((NCORE, N), jnp.float32),
    mesh=_mesh,
    scratch_types=[
        pltpu.VMEM((ROWS_A, CH), jnp.int32),
        pltpu.VMEM((ROWS_A, CH), jnp.float32),
        pltpu.VMEM((NPT,), jnp.float32),
        pltpu.VMEM_SHARED((N,), jnp.float32),
    ],
)
def _deg_partials(col_hbm, w_hbm, out_hbm, colv, wv, zbuf, deg_sh):
    c = lax.axis_index("c")
    s = lax.axis_index("s")
    tid = c * jnp.int32(NSUB) + s

    @pl.loop(0, NPT // 16)
    def _zero(i):
        zbuf[pl.ds(i * jnp.int32(16), 16)] = jnp.zeros((16,), jnp.float32)

    pltpu.sync_copy(zbuf, deg_sh.at[pl.ds(s * jnp.int32(NPT), NPT)])
    plsc.subcore_barrier()

    pltpu.sync_copy(col_hbm.at[pl.ds(tid * jnp.int32(ROWS_A), ROWS_A)], colv)
    pltpu.sync_copy(w_hbm.at[pl.ds(tid * jnp.int32(ROWS_A), ROWS_A)], wv)

    @pl.loop(0, ROWS_A)
    def _acc(j):
        pltpu.sync_copy(wv.at[j], deg_sh.at[colv.at[j]], add=True)

    plsc.subcore_barrier()
    pltpu.sync_copy(deg_sh.at[pl.ds(s * jnp.int32(NPT), NPT)],
                    out_hbm.at[c, pl.ds(s * jnp.int32(NPT), NPT)])


@functools.partial(
    pl.kernel,
    out_type=[jax.ShapeDtypeStruct((N, HHID), jnp.float32),
              jax.ShapeDtypeStruct((N, HHID), jnp.float32)],
    mesh=_mesh,
    scratch_types=[
        pltpu.VMEM((ROWS_B, CH), jnp.int32),
        pltpu.VMEM((ROWS_B, CH), jnp.int32),
        pltpu.VMEM((ROWS_B * CH,), jnp.float32),
        pltpu.VMEM((CH, HHID), jnp.float32),
        pltpu.VMEM((CH, HHID), jnp.float32),
        pltpu.VMEM((64, HHID), jnp.float32),
        pltpu.VMEM_SHARED((N, HHID), jnp.float32),
        pltpu.SemaphoreType.DMA,
        pltpu.SemaphoreType.DMA,
        pltpu.SemaphoreType.DMA,
        pltpu.SemaphoreType.DMA,
    ],
    compiler_params=_sc_params,
)
def _edge_scatter(row_hbm, col_hbm, w_hbm, g0_hbm, g1_hbm, p0_hbm, p1_hbm,
                  rowv, colv, wblk, G0, G1, zbuf, p_sh, sem0, sem1,
                  ssem0, ssem1):
    c = lax.axis_index("c")
    s = lax.axis_index("s")

    @pl.loop(0, (64 * HHID) // 16)
    def _zero(i):
        r = i // jnp.int32(HHID // 16)
        k = (i % jnp.int32(HHID // 16)) * jnp.int32(16)
        zbuf[r, pl.ds(k, 16)] = jnp.zeros((16,), jnp.float32)

    @pl.loop(0, NPT // 64)
    def _zero_sh(i):
        pltpu.sync_copy(zbuf, p_sh.at[pl.ds(s * jnp.int32(NPT) + i * jnp.int32(64), 64)])

    plsc.subcore_barrier()

    def scale(G, j):
        @pl.loop(0, CH // 16)
        def _grp(r16):
            wvec = wblk[pl.ds(j * jnp.int32(CH) + r16 * jnp.int32(16), 16)]
            rbase = r16 * jnp.int32(16)
            for l in range(16):
                wb = jnp.take_along_axis(
                    wvec, jnp.full((16,), l, jnp.int32), axis=0)
                r = rbase + jnp.int32(l)
                a = G[r, pl.ds(0, 16)] * wb
                b = G[r, pl.ds(16, 16)] * wb
                G[r, pl.ds(0, 16)] = a
                G[r, pl.ds(16, 16)] = b

    def run(g_hbm):
        def gcopy(G, sem, j):
            return pltpu.make_async_copy(g_hbm.at[rowv.at[j]], G, sem)

        def scopy(G, sem, j):
            return pltpu.make_async_copy(G, p_sh.at[colv.at[j]], sem)

        @pl.loop(0, ROWS_C // ROWS_B)
        def _blk(b):
            base = (s * jnp.int32(ROWS_C) + b * jnp.int32(ROWS_B))
            pltpu.sync_copy(row_hbm.at[pl.ds(base, ROWS_B)], rowv)
            pltpu.sync_copy(col_hbm.at[pl.ds(base, ROWS_B)], colv)
            pltpu.sync_copy(w_hbm.at[pl.ds(base * jnp.int32(CH), ROWS_B * CH)],
                            wblk)
            gcopy(G0, sem0, jnp.int32(0)).start()

            @pl.loop(0, ROWS_B // 2)
            def _main(jj):
                j0 = jj * jnp.int32(2)
                gcopy(G0, sem0, j0).wait()

                @pl.when(jj > 0)
                def _free_g1():
                    scopy(G1, ssem1, j0).wait()

                gcopy(G1, sem1, j0 + 1).start()
                scale(G0, j0)
                scopy(G0, ssem0, j0).start(add=True)
                gcopy(G1, sem1, j0 + 1).wait()

                @pl.when(jj < ROWS_B // 2 - 1)
                def _next():
                    scopy(G0, ssem0, j0).wait()
                    gcopy(G0, sem0, j0 + 2).start()

                scale(G1, j0 + 1)
                scopy(G1, ssem1, j0 + 1).start(add=True)

            scopy(G0, ssem0, jnp.int32(0)).wait()
            scopy(G1, ssem1, jnp.int32(0)).wait()

    @pl.when(c == 0)
    def _c0():
        run(g0_hbm)

    @pl.when(c == 1)
    def _c1():
        run(g1_hbm)

    plsc.subcore_barrier()

    @pl.when(c == 0)
    def _w0():
        pltpu.sync_copy(p_sh.at[pl.ds(s * jnp.int32(NPT), NPT)],
                        p0_hbm.at[pl.ds(s * jnp.int32(NPT), NPT)])

    @pl.when(c == 1)
    def _w1():
        pltpu.sync_copy(p_sh.at[pl.ds(s * jnp.int32(NPT), NPT)],
                        p1_hbm.at[pl.ds(s * jnp.int32(NPT), NPT)])


_NB = 2048
_GB = _NB // NPG


def _prep_body(x_ref, wg_ref, part_ref, g0_ref, g1_ref, dinv_ref):
    h = jnp.dot(x_ref[...], wg_ref[...], preferred_element_type=jnp.float32)
    deg = 1.0 + part_ref[0, :] + part_ref[1, :]
    dinv = jnp.where(deg > 0, lax.rsqrt(jnp.maximum(deg, 1e-12)), 0.0)
    g = h * dinv[:, None]
    g0_ref[...] = g[:, :HHID]
    g1_ref[...] = g[:, HHID:]
    dinv_ref[...] = dinv


def _head_body(p0_ref, p1_ref, g0_ref, g1_ref, dinv_ref, bg_ref,
               w1_ref, b1_ref, w2_ref, b2_ref, w3_ref, b3_ref, o_ref):
    p = jnp.concatenate([p0_ref[...], p1_ref[...]], axis=1)
    g = jnp.concatenate([g0_ref[...], g1_ref[...]], axis=1)
    agg = dinv_ref[...][:, None] * (p + g) + bg_ref[...][None, :]
    a3 = agg.reshape(_GB, NPG, HID)
    acc = jnp.zeros((_GB, HID), jnp.float32)
    for j in range(NPG):
        acc = acc + jnp.dot(a3[:, j, :], w1_ref[j],
                            preferred_element_type=jnp.float32)
    z1 = jnp.maximum(acc + b1_ref[...][None, :], 0.0)
    z2 = jnp.dot(z1, w2_ref[...], preferred_element_type=jnp.float32)
    z2 = jnp.maximum(z2 + b2_ref[...][None, :], 0.0)
    z3 = jnp.dot(z2, w3_ref[...], preferred_element_type=jnp.float32)
    z3 = z3 + b3_ref[...][None, :]
    m = jnp.max(z3, axis=-1, keepdims=True)
    e = jnp.exp(z3 - m)
    o_ref[...] = e / jnp.sum(e, axis=-1, keepdims=True)


def _prep(x, Wg, partials):
    return pl.pallas_call(
        _prep_body,
        grid=(N // _NB,),
        in_specs=[pl.BlockSpec((_NB, IN_CH), lambda i: (i, 0)),
                  pl.BlockSpec((IN_CH, HID), lambda i: (0, 0)),
                  pl.BlockSpec((NCORE, _NB), lambda i: (0, i))],
        out_specs=[pl.BlockSpec((_NB, HHID), lambda i: (i, 0)),
                   pl.BlockSpec((_NB, HHID), lambda i: (i, 0)),
                   pl.BlockSpec((_NB,), lambda i: (i,))],
        out_shape=[jax.ShapeDtypeStruct((N, HHID), jnp.float32),
                   jax.ShapeDtypeStruct((N, HHID), jnp.float32),
                   jax.ShapeDtypeStruct((N,), jnp.float32)],
    )(x, Wg, partials)


def _head(p0, p1, g0, g1, dinv, bg, W1r, b1, W2, b2, W3, b3):
    return pl.pallas_call(
        _head_body,
        grid=(N // _NB,),
        in_specs=[pl.BlockSpec((_NB, HHID), lambda i: (i, 0)),
                  pl.BlockSpec((_NB, HHID), lambda i: (i, 0)),
                  pl.BlockSpec((_NB, HHID), lambda i: (i, 0)),
                  pl.BlockSpec((_NB, HHID), lambda i: (i, 0)),
                  pl.BlockSpec((_NB,), lambda i: (i,)),
                  pl.BlockSpec((HID,), lambda i: (0,)),
                  pl.BlockSpec((NPG, HID, HID), lambda i: (0, 0, 0)),
                  pl.BlockSpec((HID,), lambda i: (0,)),
                  pl.BlockSpec((HID, HHID), lambda i: (0, 0)),
                  pl.BlockSpec((HHID,), lambda i: (0,)),
                  pl.BlockSpec((HHID, NC), lambda i: (0, 0)),
                  pl.BlockSpec((NC,), lambda i: (0,))],
        out_specs=pl.BlockSpec((_GB, NC), lambda i: (i, 0)),
        out_shape=jax.ShapeDtypeStruct((BS, NC), jnp.float32),
    )(p0, p1, g0, g1, dinv, bg, W1r, b1, W2, b2, W3, b3)


def kernel(x, edge_index, edge_attr, batch, Wg, bg, W1, b1, W2, b2, W3, b3):
    del batch
    with jax.enable_x64(False):
        x = x.astype(jnp.float32)
        ei = edge_index.astype(jnp.int32)
        row2d = ei[0].reshape(E // CH, CH)
        col2d = ei[1].reshape(E // CH, CH)
        w = edge_attr.astype(jnp.float32)
        w2d = w.reshape(E // CH, CH)
        W1r = W1.reshape(NPG, HID, HID)

        partials = _deg_partials(col2d, w2d)
        g0, g1, dinv = _prep(x, Wg, partials)
        p0, p1 = _edge_scatter(row2d, col2d, w, g0, g1)
        return _head(p0, p1, g0, g1, dinv, bg, W1r, b1, W2, b2, W3, b3)
